# initial kernel scaffold (unmeasured)
import jax
import jax.numpy as jnp
from jax import lax
from jax.experimental import pallas as pl
from jax.experimental.pallas import tpu as pltpu

N_DEV = 16
B = 2
SQS = 128
SQ = N_DEV * SQS
D = 512
H_LOC = 4
DH = 64
HD = H_LOC * DH
QC = 512


def kernel(x, Wq, Wk, Wv, Wo):
    def body(x_ref, wq_ref, wk_ref, wv_ref, wo_ref, out_ref,
             comm, xseq, qs, ks, vs, ctx, partial, sendbuf, rsbuf,
             ag_send, ag_recv, rs_send, rs_recv):
        my = lax.axis_index("i")
        left = lax.rem(my + N_DEV - 1, N_DEV)
        right = lax.rem(my + 1, N_DEV)

        barrier_sem = pltpu.get_barrier_semaphore()
        for nbr in (left, right):
            pl.semaphore_signal(barrier_sem, inc=1, device_id=(nbr,),
                                device_id_type=pl.DeviceIdType.MESH)
        pl.semaphore_wait(barrier_sem, 2)

        comm[0] = x_ref[...]
        for h in range(N_DEV - 1):
            rdma = pltpu.make_async_remote_copy(
                src_ref=comm.at[h],
                dst_ref=comm.at[h + 1],
                send_sem=ag_send.at[h % 2],
                recv_sem=ag_recv.at[h % 2],
                device_id=(right,),
                device_id_type=pl.DeviceIdType.MESH,
            )
            rdma.start()
            rdma.wait()

        for r in range(N_DEV):
            origin = lax.rem(my - r + N_DEV, N_DEV)
            for b in range(B):
                xseq[b, pl.ds(origin * SQS, SQS), :] = comm[r, b]

        posf = lax.broadcasted_iota(jnp.float32, (SQ, DH), 0)
        df = lax.broadcasted_iota(jnp.float32, (SQ, DH), 1)
        expo = (df - jnp.mod(df, 2.0)) * (1.0 / DH)
        angle = posf * jnp.exp(-jnp.log(10000.0) * expo)
        cos1 = jnp.cos(angle)
        sin1 = jnp.sin(angle)
        cos_t = jnp.concatenate([cos1] * H_LOC, axis=1)
        sin_t = jnp.concatenate([sin1] * H_LOC, axis=1)

        ii = lax.broadcasted_iota(jnp.int32, (HD, HD), 0)
        jj = lax.broadcasted_iota(jnp.int32, (HD, HD), 1)
        rot_p = (ii == jj - 1) & (jj % 2 == 1)
        rot_m = (ii == jj + 1) & (jj % 2 == 0)
        R = jnp.where(rot_p, 1.0, 0.0) + jnp.where(rot_m, -1.0, 0.0)

        for b in range(B):
            xb = xseq[b]
            qb = jnp.dot(xb, wq_ref[...], preferred_element_type=jnp.float32)
            kb = jnp.dot(xb, wk_ref[...], preferred_element_type=jnp.float32)
            vb = jnp.dot(xb, wv_ref[...], preferred_element_type=jnp.float32)
            qs[b] = qb * cos_t + jnp.dot(
                qb, R, preferred_element_type=jnp.float32) * sin_t
            ks[b] = kb * cos_t + jnp.dot(
                kb, R, preferred_element_type=jnp.float32) * sin_t
            vs[b] = vb

        for b in range(B):
            for h in range(H_LOC):
                kbh = ks[b, :, h * DH:(h + 1) * DH]
                vbh = vs[b, :, h * DH:(h + 1) * DH]
                for qc in range(SQ // QC):
                    qq = qs[b, qc * QC:(qc + 1) * QC, h * DH:(h + 1) * DH]
                    s = lax.dot_general(
                        qq, kbh, (((1,), (1,)), ((), ())),
                        preferred_element_type=jnp.float32) * 0.125
                    m = jnp.max(s, axis=1, keepdims=True)
                    w = jnp.exp(s - m)
                    w = w / jnp.sum(w, axis=1, keepdims=True)
                    cc = jnp.dot(w, vbh, preferred_element_type=jnp.float32)
                    ctx[b, qc * QC:(qc + 1) * QC, h * DH:(h + 1) * DH] = cc

        for b in range(B):
            partial[b] = jnp.dot(ctx[b], wo_ref[...],
                                 preferred_element_type=jnp.float32)

        for h in range(N_DEV - 1):
            c = lax.rem(my + (N_DEV - 1 - h), N_DEV)
            pc = partial[:, pl.ds(c * SQS, SQS), :]
            if h == 0:
                sendbuf[...] = pc
            else:
                sendbuf[...] = rsbuf[(h - 1) % 2] + pc
            rdma = pltpu.make_async_remote_copy(
                src_ref=sendbuf,
                dst_ref=rsbuf.at[h % 2],
                send_sem=rs_send.at[h % 2],
                recv_sem=rs_recv.at[h % 2],
                device_id=(right,),
                device_id_type=pl.DeviceIdType.MESH,
            )
            rdma.start()
            rdma.wait()

        out_ref[...] = rsbuf[(N_DEV - 2) % 2] + partial[:, pl.ds(my * SQS, SQS), :]

    return pl.pallas_call(
        body,
        out_shape=jax.ShapeDtypeStruct((B, SQS, D), jnp.float32),
        in_specs=[pl.BlockSpec(memory_space=pltpu.VMEM)] * 5,
        out_specs=pl.BlockSpec(memory_space=pltpu.VMEM),
        scratch_shapes=[
            pltpu.VMEM((N_DEV, B, SQS, D), jnp.float32),
            pltpu.VMEM((B, SQ, D), jnp.float32),
            pltpu.VMEM((B, SQ, HD), jnp.float32),
            pltpu.VMEM((B, SQ, HD), jnp.float32),
            pltpu.VMEM((B, SQ, HD), jnp.float32),
            pltpu.VMEM((B, SQ, HD), jnp.float32),
            pltpu.VMEM((B, SQ, D), jnp.float32),
            pltpu.VMEM((B, SQS, D), jnp.float32),
            pltpu.VMEM((2, B, SQS, D), jnp.float32),
            pltpu.SemaphoreType.DMA((2,)),
            pltpu.SemaphoreType.DMA((2,)),
            pltpu.SemaphoreType.DMA((2,)),
            pltpu.SemaphoreType.DMA((2,)),
        ],
        compiler_params=pltpu.CompilerParams(collective_id=0),
    )(x, Wq, Wk, Wv, Wo)


# baseline (device time: 299698 ns/iter reference)
import jax
import jax.numpy as jnp
from jax import lax
from jax.experimental import pallas as pl
from jax.experimental.pallas import tpu as pltpu

N_DEV = 16
B = 2
SQS = 128
SQ = N_DEV * SQS
D = 512
H_LOC = 4
DH = 64
HD = H_LOC * DH
QC = 512


def kernel(x, Wq, Wk, Wv, Wo):
    def body(x_ref, wq_ref, wk_ref, wv_ref, wo_ref, out_ref,
             comm, xseq, qs, ks, vs, ctx, partial, sendbuf, rsbuf,
             ag_send, ag_recv, rs_send, rs_recv):
        my = lax.axis_index("i")
        left = lax.rem(my + N_DEV - 1, N_DEV)
        right = lax.rem(my + 1, N_DEV)

        barrier_sem = pltpu.get_barrier_semaphore()
        for nbr in (left, right):
            pl.semaphore_signal(barrier_sem, inc=1, device_id=(nbr,),
                                device_id_type=pl.DeviceIdType.MESH)
        pl.semaphore_wait(barrier_sem, 2)

        comm[0] = x_ref[...]
        for h in range(N_DEV - 1):
            rdma = pltpu.make_async_remote_copy(
                src_ref=comm.at[h],
                dst_ref=comm.at[h + 1],
                send_sem=ag_send.at[h % 2],
                recv_sem=ag_recv.at[h % 2],
                device_id=(right,),
                device_id_type=pl.DeviceIdType.MESH,
            )
            rdma.start()
            rdma.wait()

        for r in range(N_DEV):
            origin = lax.rem(my - r + N_DEV, N_DEV)
            for b in range(B):
                xseq[b, pl.ds(origin * SQS, SQS), :] = comm[r, b]

        posf = lax.broadcasted_iota(jnp.int32, (SQ, DH), 0).astype(jnp.float32)
        di = lax.broadcasted_iota(jnp.int32, (SQ, DH), 1)
        df = (di - di % 2).astype(jnp.float32)
        expo = df * (1.0 / DH)
        angle = posf * jnp.exp(-jnp.log(10000.0) * expo)
        cos1 = jnp.cos(angle)
        sin1 = jnp.sin(angle)
        cos_t = jnp.concatenate([cos1] * H_LOC, axis=1)
        sin_t = jnp.concatenate([sin1] * H_LOC, axis=1)

        ii = lax.broadcasted_iota(jnp.int32, (HD, HD), 0)
        jj = lax.broadcasted_iota(jnp.int32, (HD, HD), 1)
        rot_p = (ii == jj - 1) & (jj % 2 == 1)
        rot_m = (ii == jj + 1) & (jj % 2 == 0)
        R = jnp.where(rot_p, 1.0, 0.0) + jnp.where(rot_m, -1.0, 0.0)

        for b in range(B):
            xb = xseq[b]
            qb = jnp.dot(xb, wq_ref[...], preferred_element_type=jnp.float32)
            kb = jnp.dot(xb, wk_ref[...], preferred_element_type=jnp.float32)
            vb = jnp.dot(xb, wv_ref[...], preferred_element_type=jnp.float32)
            qs[b] = qb * cos_t + jnp.dot(
                qb, R, preferred_element_type=jnp.float32) * sin_t
            ks[b] = kb * cos_t + jnp.dot(
                kb, R, preferred_element_type=jnp.float32) * sin_t
            vs[b] = vb

        for b in range(B):
            for h in range(H_LOC):
                kbh = ks[b, :, h * DH:(h + 1) * DH]
                vbh = vs[b, :, h * DH:(h + 1) * DH]
                for qc in range(SQ // QC):
                    qq = qs[b, qc * QC:(qc + 1) * QC, h * DH:(h + 1) * DH]
                    s = lax.dot_general(
                        qq, kbh, (((1,), (1,)), ((), ())),
                        preferred_element_type=jnp.float32) * 0.125
                    m = jnp.max(s, axis=1, keepdims=True)
                    w = jnp.exp(s - m)
                    w = w / jnp.sum(w, axis=1, keepdims=True)
                    cc = jnp.dot(w, vbh, preferred_element_type=jnp.float32)
                    ctx[b, qc * QC:(qc + 1) * QC, h * DH:(h + 1) * DH] = cc

        for b in range(B):
            partial[b] = jnp.dot(ctx[b], wo_ref[...],
                                 preferred_element_type=jnp.float32)

        for h in range(N_DEV - 1):
            c = lax.rem(my + (N_DEV - 1 - h), N_DEV)
            pc = partial[:, pl.ds(c * SQS, SQS), :]
            if h == 0:
                sendbuf[...] = pc
            else:
                sendbuf[...] = rsbuf[(h - 1) % 2] + pc
            rdma = pltpu.make_async_remote_copy(
                src_ref=sendbuf,
                dst_ref=rsbuf.at[h % 2],
                send_sem=rs_send.at[h % 2],
                recv_sem=rs_recv.at[h % 2],
                device_id=(right,),
                device_id_type=pl.DeviceIdType.MESH,
            )
            rdma.start()
            rdma.wait()

        out_ref[...] = rsbuf[(N_DEV - 2) % 2] + partial[:, pl.ds(my * SQS, SQS), :]

    return pl.pallas_call(
        body,
        out_shape=jax.ShapeDtypeStruct((B, SQS, D), jnp.float32),
        in_specs=[pl.BlockSpec(memory_space=pltpu.VMEM)] * 5,
        out_specs=pl.BlockSpec(memory_space=pltpu.VMEM),
        scratch_shapes=[
            pltpu.VMEM((N_DEV, B, SQS, D), jnp.float32),
            pltpu.VMEM((B, SQ, D), jnp.float32),
            pltpu.VMEM((B, SQ, HD), jnp.float32),
            pltpu.VMEM((B, SQ, HD), jnp.float32),
            pltpu.VMEM((B, SQ, HD), jnp.float32),
            pltpu.VMEM((B, SQ, HD), jnp.float32),
            pltpu.VMEM((B, SQ, D), jnp.float32),
            pltpu.VMEM((B, SQS, D), jnp.float32),
            pltpu.VMEM((2, B, SQS, D), jnp.float32),
            pltpu.SemaphoreType.DMA((2,)),
            pltpu.SemaphoreType.DMA((2,)),
            pltpu.SemaphoreType.DMA((2,)),
            pltpu.SemaphoreType.DMA((2,)),
        ],
        compiler_params=pltpu.CompilerParams(
            collective_id=0, vmem_limit_bytes=100 * 1024 * 1024),
    )(x, Wq, Wk, Wv, Wo)


# device time: 214129 ns/iter; 1.3996x vs baseline; 1.3996x over previous
import jax
import jax.numpy as jnp
from jax import lax
from jax.experimental import pallas as pl
from jax.experimental.pallas import tpu as pltpu

N_DEV = 16
HALF = N_DEV // 2
B = 2
SQS = 128
SQ = N_DEV * SQS
D = 512
H_LOC = 4
DH = 64
HD = H_LOC * DH
QC = 512

_F32 = jnp.float32


def kernel(x, Wq, Wk, Wv, Wo):
    def body(x_ref, wq_ref, wk_ref, wv_ref, wo_ref, out_ref,
             cw, ccw, cos_ref, sin_ref, qs, ks, vs, ctx, partial,
             cw_sb, ccw_sb, cw_rb, ccw_rb,
             ag_cw_s, ag_cw_r, ag_ccw_s, ag_ccw_r,
             rs_cw_s, rs_cw_r, rs_ccw_s, rs_ccw_r):
        my = lax.axis_index("i")
        left = lax.rem(my + N_DEV - 1, N_DEV)
        right = lax.rem(my + 1, N_DEV)

        barrier_sem = pltpu.get_barrier_semaphore()
        for nbr in (left, right):
            pl.semaphore_signal(barrier_sem, inc=1, device_id=(nbr,),
                                device_id_type=pl.DeviceIdType.MESH)
        pl.semaphore_wait(barrier_sem, 2)

        posf = lax.broadcasted_iota(jnp.int32, (SQ, DH), 0).astype(_F32)
        di = lax.broadcasted_iota(jnp.int32, (SQ, DH), 1)
        expo = (di - di % 2).astype(_F32) * (1.0 / DH)
        angle = posf * jnp.exp(-jnp.log(10000.0) * expo)
        cos_ref[...] = jnp.cos(angle)
        sin_ref[...] = jnp.sin(angle)

        ii = lax.broadcasted_iota(jnp.int32, (HD, HD), 0)
        jj = lax.broadcasted_iota(jnp.int32, (HD, HD), 1)
        R = (jnp.where((ii == jj - 1) & (jj % 2 == 1), 1.0, 0.0)
             + jnp.where((ii == jj + 1) & (jj % 2 == 0), -1.0, 0.0))

        def qkv_chunk(src_ref, r, origin):
            start = origin * SQS
            cosc = jnp.concatenate(
                [cos_ref[pl.ds(start, SQS), :]] * H_LOC, axis=1)
            sinc = jnp.concatenate(
                [sin_ref[pl.ds(start, SQS), :]] * H_LOC, axis=1)
            for b in range(B):
                xcb = src_ref[r, b]
                qc = jnp.dot(xcb, wq_ref[...], preferred_element_type=_F32)
                kc = jnp.dot(xcb, wk_ref[...], preferred_element_type=_F32)
                vc = jnp.dot(xcb, wv_ref[...], preferred_element_type=_F32)
                qs[b, pl.ds(start, SQS), :] = qc * cosc + jnp.dot(
                    qc, R, preferred_element_type=_F32) * sinc
                ks[b, pl.ds(start, SQS), :] = kc * cosc + jnp.dot(
                    kc, R, preferred_element_type=_F32) * sinc
                vs[b, pl.ds(start, SQS), :] = vc

        cw[0] = x_ref[...]
        ccw[0] = x_ref[...]
        for h in range(HALF):
            r_cw = pltpu.make_async_remote_copy(
                src_ref=cw.at[h], dst_ref=cw.at[h + 1],
                send_sem=ag_cw_s.at[h % 2], recv_sem=ag_cw_r.at[h % 2],
                device_id=(right,), device_id_type=pl.DeviceIdType.MESH)
            r_cw.start()
            if h < HALF - 1:
                r_ccw = pltpu.make_async_remote_copy(
                    src_ref=ccw.at[h], dst_ref=ccw.at[h + 1],
                    send_sem=ag_ccw_s.at[h % 2], recv_sem=ag_ccw_r.at[h % 2],
                    device_id=(left,), device_id_type=pl.DeviceIdType.MESH)
                r_ccw.start()
            qkv_chunk(cw, h, lax.rem(my - h + N_DEV, N_DEV))
            if h >= 1:
                qkv_chunk(ccw, h, lax.rem(my + h, N_DEV))
            r_cw.wait()
            if h < HALF - 1:
                r_ccw.wait()
        qkv_chunk(cw, HALF, lax.rem(my - HALF + N_DEV, N_DEV))

        for b in range(B):
            for h in range(H_LOC):
                kbh = ks[b, :, h * DH:(h + 1) * DH]
                vbh = vs[b, :, h * DH:(h + 1) * DH]
                for qc in range(SQ // QC):
                    qq = qs[b, qc * QC:(qc + 1) * QC, h * DH:(h + 1) * DH]
                    s = lax.dot_general(
                        qq, kbh, (((1,), (1,)), ((), ())),
                        preferred_element_type=_F32) * 0.125
                    m = jnp.max(s, axis=1, keepdims=True)
                    w = jnp.exp(s - m)
                    w = w / jnp.sum(w, axis=1, keepdims=True)
                    cc = jnp.dot(w, vbh, preferred_element_type=_F32)
                    ctx[b, qc * QC:(qc + 1) * QC, h * DH:(h + 1) * DH] = cc

        for b in range(B):
            partial[b] = jnp.dot(ctx[b], wo_ref[...],
                                 preferred_element_type=_F32)

        for h in range(HALF):
            c1 = lax.rem(my + HALF - h + N_DEV, N_DEV)
            p1 = partial[:, pl.ds(c1 * SQS, SQS), :]
            if h == 0:
                cw_sb[...] = p1
            else:
                cw_sb[...] = cw_rb[(h - 1) % 2] + p1
            r_cw = pltpu.make_async_remote_copy(
                src_ref=cw_sb, dst_ref=cw_rb.at[h % 2],
                send_sem=rs_cw_s.at[h % 2], recv_sem=rs_cw_r.at[h % 2],
                device_id=(right,), device_id_type=pl.DeviceIdType.MESH)
            r_cw.start()
            if h < HALF - 1:
                c2 = lax.rem(my - (HALF - 1) + h + N_DEV, N_DEV)
                p2 = partial[:, pl.ds(c2 * SQS, SQS), :]
                if h == 0:
                    ccw_sb[...] = p2
                else:
                    ccw_sb[...] = ccw_rb[(h - 1) % 2] + p2
                r_ccw = pltpu.make_async_remote_copy(
                    src_ref=ccw_sb, dst_ref=ccw_rb.at[h % 2],
                    send_sem=rs_ccw_s.at[h % 2], recv_sem=rs_ccw_r.at[h % 2],
                    device_id=(left,), device_id_type=pl.DeviceIdType.MESH)
                r_ccw.start()
            r_cw.wait()
            if h < HALF - 1:
                r_ccw.wait()

        out_ref[...] = (cw_rb[(HALF - 1) % 2] + ccw_rb[(HALF - 2) % 2]
                        + partial[:, pl.ds(my * SQS, SQS), :])

    return pl.pallas_call(
        body,
        out_shape=jax.ShapeDtypeStruct((B, SQS, D), _F32),
        in_specs=[pl.BlockSpec(memory_space=pltpu.VMEM)] * 5,
        out_specs=pl.BlockSpec(memory_space=pltpu.VMEM),
        scratch_shapes=[
            pltpu.VMEM((HALF + 1, B, SQS, D), _F32),
            pltpu.VMEM((HALF, B, SQS, D), _F32),
            pltpu.VMEM((SQ, DH), _F32),
            pltpu.VMEM((SQ, DH), _F32),
            pltpu.VMEM((B, SQ, HD), _F32),
            pltpu.VMEM((B, SQ, HD), _F32),
            pltpu.VMEM((B, SQ, HD), _F32),
            pltpu.VMEM((B, SQ, HD), _F32),
            pltpu.VMEM((B, SQ, D), _F32),
            pltpu.VMEM((B, SQS, D), _F32),
            pltpu.VMEM((B, SQS, D), _F32),
            pltpu.VMEM((2, B, SQS, D), _F32),
            pltpu.VMEM((2, B, SQS, D), _F32),
            pltpu.SemaphoreType.DMA((2,)),
            pltpu.SemaphoreType.DMA((2,)),
            pltpu.SemaphoreType.DMA((2,)),
            pltpu.SemaphoreType.DMA((2,)),
            pltpu.SemaphoreType.DMA((2,)),
            pltpu.SemaphoreType.DMA((2,)),
            pltpu.SemaphoreType.DMA((2,)),
            pltpu.SemaphoreType.DMA((2,)),
        ],
        compiler_params=pltpu.CompilerParams(
            collective_id=0, vmem_limit_bytes=100 * 1024 * 1024),
    )(x, Wq, Wk, Wv, Wo)


# device time: 191527 ns/iter; 1.5648x vs baseline; 1.1180x over previous
import jax
import jax.numpy as jnp
from jax import lax
from jax.experimental import pallas as pl
from jax.experimental.pallas import tpu as pltpu

N_DEV = 16
HALF = N_DEV // 2
B = 2
SQS = 128
SQ = N_DEV * SQS
D = 512
H_LOC = 4
DH = 64
HD = H_LOC * DH
QC = 512

_F32 = jnp.float32


def kernel(x, Wq, Wk, Wv, Wo):
    def body(x_ref, wq_ref, wk_ref, wv_ref, wo_ref, out_ref,
             cw, ccw, cos_ref, sin_ref, qs, ks, vs,
             cw_sb, ccw_sb, cw_rb, ccw_rb,
             ag_cw_s, ag_cw_r, ag_ccw_s, ag_ccw_r,
             rs_cw_s, rs_cw_r, rs_ccw_s, rs_ccw_r):
        my = lax.axis_index("i")
        left = lax.rem(my + N_DEV - 1, N_DEV)
        right = lax.rem(my + 1, N_DEV)

        barrier_sem = pltpu.get_barrier_semaphore()
        for nbr in (left, right):
            pl.semaphore_signal(barrier_sem, inc=1, device_id=(nbr,),
                                device_id_type=pl.DeviceIdType.MESH)
        pl.semaphore_wait(barrier_sem, 2)

        posf = lax.broadcasted_iota(jnp.int32, (SQ, DH), 0).astype(_F32)
        di = lax.broadcasted_iota(jnp.int32, (SQ, DH), 1)
        expo = (di - di % 2).astype(_F32) * (1.0 / DH)
        angle = posf * jnp.exp(-jnp.log(10000.0) * expo)
        cos_ref[...] = jnp.cos(angle)
        sin_ref[...] = jnp.sin(angle)

        ii = lax.broadcasted_iota(jnp.int32, (HD, HD), 0)
        jj = lax.broadcasted_iota(jnp.int32, (HD, HD), 1)
        R = (jnp.where((ii == jj - 1) & (jj % 2 == 1), 1.0, 0.0)
             + jnp.where((ii == jj + 1) & (jj % 2 == 0), -1.0, 0.0))

        def qkv_chunk(src_ref, r, origin):
            start = origin * SQS
            cosc = jnp.concatenate(
                [cos_ref[pl.ds(start, SQS), :]] * H_LOC, axis=1)
            sinc = jnp.concatenate(
                [sin_ref[pl.ds(start, SQS), :]] * H_LOC, axis=1)
            for b in range(B):
                xcb = src_ref[r, b]
                qc = jnp.dot(xcb, wq_ref[...], preferred_element_type=_F32)
                kc = jnp.dot(xcb, wk_ref[...], preferred_element_type=_F32)
                vc = jnp.dot(xcb, wv_ref[...], preferred_element_type=_F32)
                qs[b, pl.ds(start, SQS), :] = qc * cosc + jnp.dot(
                    qc, R, preferred_element_type=_F32) * sinc
                ks[b, pl.ds(start, SQS), :] = kc * cosc + jnp.dot(
                    kc, R, preferred_element_type=_F32) * sinc
                vs[b, pl.ds(start, SQS), :] = vc

        cw[0] = x_ref[...]
        ccw[0] = x_ref[...]
        for h in range(HALF):
            r_cw = pltpu.make_async_remote_copy(
                src_ref=cw.at[h], dst_ref=cw.at[h + 1],
                send_sem=ag_cw_s.at[h % 2], recv_sem=ag_cw_r.at[h % 2],
                device_id=(right,), device_id_type=pl.DeviceIdType.MESH)
            r_cw.start()
            if h < HALF - 1:
                r_ccw = pltpu.make_async_remote_copy(
                    src_ref=ccw.at[h], dst_ref=ccw.at[h + 1],
                    send_sem=ag_ccw_s.at[h % 2], recv_sem=ag_ccw_r.at[h % 2],
                    device_id=(left,), device_id_type=pl.DeviceIdType.MESH)
                r_ccw.start()
            qkv_chunk(cw, h, lax.rem(my - h + N_DEV, N_DEV))
            if h >= 1:
                qkv_chunk(ccw, h, lax.rem(my + h, N_DEV))
            r_cw.wait()
            if h < HALF - 1:
                r_ccw.wait()
        qkv_chunk(cw, HALF, lax.rem(my - HALF + N_DEV, N_DEV))

        def attn_proj_chunk(c):
            outs = []
            for b in range(B):
                ccs = []
                for h in range(H_LOC):
                    qq = qs[b, pl.ds(c * SQS, SQS), h * DH:(h + 1) * DH]
                    kbh = ks[b, :, h * DH:(h + 1) * DH]
                    vbh = vs[b, :, h * DH:(h + 1) * DH]
                    s = lax.dot_general(
                        qq, kbh, (((1,), (1,)), ((), ())),
                        preferred_element_type=_F32) * 0.125
                    m = jnp.max(s, axis=1, keepdims=True)
                    w = jnp.exp(s - m)
                    w = w / jnp.sum(w, axis=1, keepdims=True)
                    ccs.append(jnp.dot(w, vbh, preferred_element_type=_F32))
                ctxb = jnp.concatenate(ccs, axis=1)
                outs.append(jnp.dot(ctxb, wo_ref[...],
                                    preferred_element_type=_F32))
            return outs

        pp_cw = attn_proj_chunk(lax.rem(my + HALF, N_DEV))
        pp_ccw = attn_proj_chunk(lax.rem(my - (HALF - 1) + N_DEV, N_DEV))
        pp_own = None
        for h in range(HALF):
            for b in range(B):
                if h == 0:
                    cw_sb[b] = pp_cw[b]
                else:
                    cw_sb[b] = cw_rb[(h - 1) % 2, b] + pp_cw[b]
            r_cw = pltpu.make_async_remote_copy(
                src_ref=cw_sb, dst_ref=cw_rb.at[h % 2],
                send_sem=rs_cw_s.at[h % 2], recv_sem=rs_cw_r.at[h % 2],
                device_id=(right,), device_id_type=pl.DeviceIdType.MESH)
            r_cw.start()
            if h < HALF - 1:
                for b in range(B):
                    if h == 0:
                        ccw_sb[b] = pp_ccw[b]
                    else:
                        ccw_sb[b] = ccw_rb[(h - 1) % 2, b] + pp_ccw[b]
                r_ccw = pltpu.make_async_remote_copy(
                    src_ref=ccw_sb, dst_ref=ccw_rb.at[h % 2],
                    send_sem=rs_ccw_s.at[h % 2], recv_sem=rs_ccw_r.at[h % 2],
                    device_id=(left,), device_id_type=pl.DeviceIdType.MESH)
                r_ccw.start()
            if h < HALF - 1:
                pp_cw = attn_proj_chunk(lax.rem(my + HALF - h - 1 + N_DEV,
                                                N_DEV))
            if h < HALF - 2:
                pp_ccw = attn_proj_chunk(lax.rem(my - (HALF - 1) + h + 1
                                                 + N_DEV, N_DEV))
            if h == HALF - 1:
                pp_own = attn_proj_chunk(my)
            r_cw.wait()
            if h < HALF - 1:
                r_ccw.wait()

        for b in range(B):
            out_ref[b] = (cw_rb[(HALF - 1) % 2, b]
                          + ccw_rb[(HALF - 2) % 2, b] + pp_own[b])

    return pl.pallas_call(
        body,
        out_shape=jax.ShapeDtypeStruct((B, SQS, D), _F32),
        in_specs=[pl.BlockSpec(memory_space=pltpu.VMEM)] * 5,
        out_specs=pl.BlockSpec(memory_space=pltpu.VMEM),
        scratch_shapes=[
            pltpu.VMEM((HALF + 1, B, SQS, D), _F32),
            pltpu.VMEM((HALF, B, SQS, D), _F32),
            pltpu.VMEM((SQ, DH), _F32),
            pltpu.VMEM((SQ, DH), _F32),
            pltpu.VMEM((B, SQ, HD), _F32),
            pltpu.VMEM((B, SQ, HD), _F32),
            pltpu.VMEM((B, SQ, HD), _F32),
            pltpu.VMEM((B, SQS, D), _F32),
            pltpu.VMEM((B, SQS, D), _F32),
            pltpu.VMEM((2, B, SQS, D), _F32),
            pltpu.VMEM((2, B, SQS, D), _F32),
            pltpu.SemaphoreType.DMA((2,)),
            pltpu.SemaphoreType.DMA((2,)),
            pltpu.SemaphoreType.DMA((2,)),
            pltpu.SemaphoreType.DMA((2,)),
            pltpu.SemaphoreType.DMA((2,)),
            pltpu.SemaphoreType.DMA((2,)),
            pltpu.SemaphoreType.DMA((2,)),
            pltpu.SemaphoreType.DMA((2,)),
        ],
        compiler_params=pltpu.CompilerParams(
            collective_id=0, vmem_limit_bytes=100 * 1024 * 1024),
    )(x, Wq, Wk, Wv, Wo)


# device time: 170119 ns/iter; 1.7617x vs baseline; 1.1258x over previous
import jax
import jax.numpy as jnp
from jax import lax
from jax.experimental import pallas as pl
from jax.experimental.pallas import tpu as pltpu

N_DEV = 16
HALF = N_DEV // 2
B = 2
SQS = 128
SQ = N_DEV * SQS
D = 512
H_LOC = 4
DH = 64
HD = H_LOC * DH
_F32 = jnp.float32
_BF16 = jnp.bfloat16


def kernel(x, Wq, Wk, Wv, Wo):
    def body(x_ref, wq_ref, wk_ref, wv_ref, wo_ref, out_ref,
             cw, ccw, cos_ref, sin_ref, qs, ks, vs,
             wqb, wkb, wvb, wob,
             cw_sb, ccw_sb, cw_rb, ccw_rb,
             ag_cw_s, ag_cw_r, ag_ccw_s, ag_ccw_r,
             rs_cw_s, rs_cw_r, rs_ccw_s, rs_ccw_r):
        my = lax.axis_index("i")
        left = lax.rem(my + N_DEV - 1, N_DEV)
        right = lax.rem(my + 1, N_DEV)

        barrier_sem = pltpu.get_barrier_semaphore()
        for nbr in (left, right):
            pl.semaphore_signal(barrier_sem, inc=1, device_id=(nbr,),
                                device_id_type=pl.DeviceIdType.MESH)
        pl.semaphore_wait(barrier_sem, 2)

        posf = lax.broadcasted_iota(jnp.int32, (SQ, DH), 0).astype(_F32)
        di = lax.broadcasted_iota(jnp.int32, (SQ, DH), 1)
        expo = (di - di % 2).astype(_F32) * (1.0 / DH)
        angle = posf * jnp.exp(-jnp.log(10000.0) * expo)
        cos_ref[...] = jnp.cos(angle)
        sin_ref[...] = jnp.sin(angle)

        ii = lax.broadcasted_iota(jnp.int32, (HD, HD), 0)
        jj = lax.broadcasted_iota(jnp.int32, (HD, HD), 1)
        R = (jnp.where((ii == jj - 1) & (jj % 2 == 1), 1.0, 0.0)
             + jnp.where((ii == jj + 1) & (jj % 2 == 0), -1.0, 0.0))

        wqb[...] = wq_ref[...].astype(_BF16)
        wkb[...] = wk_ref[...].astype(_BF16)
        wvb[...] = wv_ref[...].astype(_BF16)
        wob[...] = wo_ref[...].astype(_BF16)

        def qkv_chunk(src_ref, r, origin):
            start = origin * SQS
            cosc = jnp.concatenate(
                [cos_ref[pl.ds(start, SQS), :]] * H_LOC, axis=1)
            sinc = jnp.concatenate(
                [sin_ref[pl.ds(start, SQS), :]] * H_LOC, axis=1)
            for b in range(B):
                xcb = src_ref[r, b]
                qc = jnp.dot(xcb, wqb[...], preferred_element_type=_F32)
                kc = jnp.dot(xcb, wkb[...], preferred_element_type=_F32)
                vc = jnp.dot(xcb, wvb[...], preferred_element_type=_F32)
                qs[b, pl.ds(start, SQS), :] = (qc * cosc + jnp.dot(
                    qc, R, preferred_element_type=_F32) * sinc).astype(_BF16)
                ks[b, pl.ds(start, SQS), :] = (kc * cosc + jnp.dot(
                    kc, R, preferred_element_type=_F32) * sinc).astype(_BF16)
                vs[b, pl.ds(start, SQS), :] = vc.astype(_BF16)

        cw[0] = x_ref[...].astype(_BF16)
        ccw[0] = cw[0]
        for h in range(HALF):
            r_cw = pltpu.make_async_remote_copy(
                src_ref=cw.at[h], dst_ref=cw.at[h + 1],
                send_sem=ag_cw_s.at[h % 2], recv_sem=ag_cw_r.at[h % 2],
                device_id=(right,), device_id_type=pl.DeviceIdType.MESH)
            r_cw.start()
            if h < HALF - 1:
                r_ccw = pltpu.make_async_remote_copy(
                    src_ref=ccw.at[h], dst_ref=ccw.at[h + 1],
                    send_sem=ag_ccw_s.at[h % 2], recv_sem=ag_ccw_r.at[h % 2],
                    device_id=(left,), device_id_type=pl.DeviceIdType.MESH)
                r_ccw.start()
            qkv_chunk(cw, h, lax.rem(my - h + N_DEV, N_DEV))
            if h >= 1:
                qkv_chunk(ccw, h, lax.rem(my + h, N_DEV))
            r_cw.wait()
            if h < HALF - 1:
                r_ccw.wait()
        qkv_chunk(cw, HALF, lax.rem(my - HALF + N_DEV, N_DEV))

        def attn_proj_chunk(c):
            outs = []
            for b in range(B):
                ccs = []
                for h in range(H_LOC):
                    qq = qs[b, pl.ds(c * SQS, SQS), h * DH:(h + 1) * DH]
                    kbh = ks[b, :, h * DH:(h + 1) * DH]
                    vbh = vs[b, :, h * DH:(h + 1) * DH]
                    s = lax.dot_general(
                        qq, kbh, (((1,), (1,)), ((), ())),
                        preferred_element_type=_F32) * 0.125
                    m = jnp.max(s, axis=1, keepdims=True)
                    w = jnp.exp(s - m)
                    w = (w / jnp.sum(w, axis=1, keepdims=True)).astype(_BF16)
                    ccs.append(jnp.dot(w, vbh, preferred_element_type=_F32))
                ctxb = jnp.concatenate(ccs, axis=1).astype(_BF16)
                outs.append(jnp.dot(ctxb, wob[...],
                                    preferred_element_type=_F32))
            return outs

        pp_cw = attn_proj_chunk(lax.rem(my + HALF, N_DEV))
        pp_ccw = attn_proj_chunk(lax.rem(my - (HALF - 1) + N_DEV, N_DEV))
        pp_own = None
        for h in range(HALF):
            for b in range(B):
                if h == 0:
                    cw_sb[b] = pp_cw[b]
                else:
                    cw_sb[b] = cw_rb[(h - 1) % 2, b] + pp_cw[b]
            r_cw = pltpu.make_async_remote_copy(
                src_ref=cw_sb, dst_ref=cw_rb.at[h % 2],
                send_sem=rs_cw_s.at[h % 2], recv_sem=rs_cw_r.at[h % 2],
                device_id=(right,), device_id_type=pl.DeviceIdType.MESH)
            r_cw.start()
            if h < HALF - 1:
                for b in range(B):
                    if h == 0:
                        ccw_sb[b] = pp_ccw[b]
                    else:
                        ccw_sb[b] = ccw_rb[(h - 1) % 2, b] + pp_ccw[b]
                r_ccw = pltpu.make_async_remote_copy(
                    src_ref=ccw_sb, dst_ref=ccw_rb.at[h % 2],
                    send_sem=rs_ccw_s.at[h % 2], recv_sem=rs_ccw_r.at[h % 2],
                    device_id=(left,), device_id_type=pl.DeviceIdType.MESH)
                r_ccw.start()
            if h < HALF - 1:
                pp_cw = attn_proj_chunk(lax.rem(my + HALF - h - 1 + N_DEV,
                                                N_DEV))
            if h < HALF - 2:
                pp_ccw = attn_proj_chunk(lax.rem(my - (HALF - 1) + h + 1
                                                 + N_DEV, N_DEV))
            if h == HALF - 1:
                pp_own = attn_proj_chunk(my)
            r_cw.wait()
            if h < HALF - 1:
                r_ccw.wait()

        for b in range(B):
            out_ref[b] = (cw_rb[(HALF - 1) % 2, b]
                          + ccw_rb[(HALF - 2) % 2, b] + pp_own[b])

    return pl.pallas_call(
        body,
        out_shape=jax.ShapeDtypeStruct((B, SQS, D), _F32),
        in_specs=[pl.BlockSpec(memory_space=pltpu.VMEM)] * 5,
        out_specs=pl.BlockSpec(memory_space=pltpu.VMEM),
        scratch_shapes=[
            pltpu.VMEM((HALF + 1, B, SQS, D), _BF16),
            pltpu.VMEM((HALF, B, SQS, D), _BF16),
            pltpu.VMEM((SQ, DH), _F32),
            pltpu.VMEM((SQ, DH), _F32),
            pltpu.VMEM((B, SQ, HD), _BF16),
            pltpu.VMEM((B, SQ, HD), _BF16),
            pltpu.VMEM((B, SQ, HD), _BF16),
            pltpu.VMEM((D, HD), _BF16),
            pltpu.VMEM((D, HD), _BF16),
            pltpu.VMEM((D, HD), _BF16),
            pltpu.VMEM((HD, D), _BF16),
            pltpu.VMEM((B, SQS, D), _F32),
            pltpu.VMEM((B, SQS, D), _F32),
            pltpu.VMEM((2, B, SQS, D), _F32),
            pltpu.VMEM((2, B, SQS, D), _F32),
            pltpu.SemaphoreType.DMA((2,)),
            pltpu.SemaphoreType.DMA((2,)),
            pltpu.SemaphoreType.DMA((2,)),
            pltpu.SemaphoreType.DMA((2,)),
            pltpu.SemaphoreType.DMA((2,)),
            pltpu.SemaphoreType.DMA((2,)),
            pltpu.SemaphoreType.DMA((2,)),
            pltpu.SemaphoreType.DMA((2,)),
        ],
        compiler_params=pltpu.CompilerParams(
            collective_id=0, vmem_limit_bytes=100 * 1024 * 1024),
    )(x, Wq, Wk, Wv, Wo)


# device time: 166241 ns/iter; 1.8028x vs baseline; 1.0233x over previous
import jax
import jax.numpy as jnp
from jax import lax
from jax.experimental import pallas as pl
from jax.experimental.pallas import tpu as pltpu

N_DEV = 16
HALF = N_DEV // 2
B = 2
SQS = 128
SQ = N_DEV * SQS
D = 512
H_LOC = 4
DH = 64
HD = H_LOC * DH
_F32 = jnp.float32
_BF16 = jnp.bfloat16


def kernel(x, Wq, Wk, Wv, Wo):
    def body(x_ref, wq_ref, wk_ref, wv_ref, wo_ref, out_ref,
             cw, ccw, cos_ref, sin_ref, qs, ks, vs,
             wqb, wkb, wvb, wob,
             cw_sb, ccw_sb, cw_rb, ccw_rb,
             ag_cw_s, ag_cw_r, ag_ccw_s, ag_ccw_r,
             rs_cw_s, rs_cw_r, rs_ccw_s, rs_ccw_r):
        my = lax.axis_index("i")
        left = lax.rem(my + N_DEV - 1, N_DEV)
        right = lax.rem(my + 1, N_DEV)

        barrier_sem = pltpu.get_barrier_semaphore()
        for nbr in (left, right):
            pl.semaphore_signal(barrier_sem, inc=1, device_id=(nbr,),
                                device_id_type=pl.DeviceIdType.MESH)
        pl.semaphore_wait(barrier_sem, 2)

        posf = lax.broadcasted_iota(jnp.int32, (SQ, DH), 0).astype(_F32)
        di = lax.broadcasted_iota(jnp.int32, (SQ, DH), 1)
        expo = (di - di % 2).astype(_F32) * (1.0 / DH)
        angle = posf * jnp.exp(-jnp.log(10000.0) * expo)
        cos_ref[...] = jnp.cos(angle)
        sin_ref[...] = jnp.sin(angle)

        ii = lax.broadcasted_iota(jnp.int32, (HD, HD), 0)
        jj = lax.broadcasted_iota(jnp.int32, (HD, HD), 1)
        R = (jnp.where((ii == jj - 1) & (jj % 2 == 1), 1.0, 0.0)
             + jnp.where((ii == jj + 1) & (jj % 2 == 0), -1.0, 0.0))

        wqb[...] = wq_ref[...].astype(_BF16)
        wkb[...] = wk_ref[...].astype(_BF16)
        wvb[...] = wv_ref[...].astype(_BF16)
        wob[...] = wo_ref[...].astype(_BF16)

        def qkv_chunk(src_ref, r, origin):
            start = origin * SQS
            cosc = jnp.concatenate(
                [cos_ref[pl.ds(start, SQS), :]] * H_LOC, axis=1)
            sinc = jnp.concatenate(
                [sin_ref[pl.ds(start, SQS), :]] * H_LOC, axis=1)
            for b in range(B):
                xcb = src_ref[r, b]
                qc = jnp.dot(xcb, wqb[...], preferred_element_type=_F32)
                kc = jnp.dot(xcb, wkb[...], preferred_element_type=_F32)
                vc = jnp.dot(xcb, wvb[...], preferred_element_type=_F32)
                qs[b, pl.ds(start, SQS), :] = (qc * cosc + jnp.dot(
                    qc, R, preferred_element_type=_F32) * sinc).astype(_BF16)
                ks[b, pl.ds(start, SQS), :] = (kc * cosc + jnp.dot(
                    kc, R, preferred_element_type=_F32) * sinc).astype(_BF16)
                vs[b, pl.ds(start, SQS), :] = vc.astype(_BF16)

        cw[0] = x_ref[...].astype(_BF16)
        ccw[0] = cw[0]
        for h in range(HALF):
            r_cw = pltpu.make_async_remote_copy(
                src_ref=cw.at[h], dst_ref=cw.at[h + 1],
                send_sem=ag_cw_s.at[h % 2], recv_sem=ag_cw_r.at[h % 2],
                device_id=(right,), device_id_type=pl.DeviceIdType.MESH)
            r_cw.start()
            if h < HALF - 1:
                r_ccw = pltpu.make_async_remote_copy(
                    src_ref=ccw.at[h], dst_ref=ccw.at[h + 1],
                    send_sem=ag_ccw_s.at[h % 2], recv_sem=ag_ccw_r.at[h % 2],
                    device_id=(left,), device_id_type=pl.DeviceIdType.MESH)
                r_ccw.start()
            qkv_chunk(cw, h, lax.rem(my - h + N_DEV, N_DEV))
            if h >= 1:
                qkv_chunk(ccw, h, lax.rem(my + h, N_DEV))
            r_cw.wait()
            if h < HALF - 1:
                r_ccw.wait()
        qkv_chunk(cw, HALF, lax.rem(my - HALF + N_DEV, N_DEV))

        def attn_proj_chunk(c):
            outs = []
            for b in range(B):
                ccs = []
                for h in range(H_LOC):
                    qq = qs[b, pl.ds(c * SQS, SQS), h * DH:(h + 1) * DH]
                    kbh = ks[b, :, h * DH:(h + 1) * DH]
                    vbh = vs[b, :, h * DH:(h + 1) * DH]
                    s = lax.dot_general(
                        qq, kbh, (((1,), (1,)), ((), ())),
                        preferred_element_type=_F32) * 0.125
                    m = jnp.max(s, axis=1, keepdims=True)
                    w = jnp.exp(s - m)
                    w = (w / jnp.sum(w, axis=1, keepdims=True)).astype(_BF16)
                    ccs.append(jnp.dot(w, vbh, preferred_element_type=_F32))
                ctxb = jnp.concatenate(ccs, axis=1).astype(_BF16)
                outs.append(jnp.dot(ctxb, wob[...],
                                    preferred_element_type=_F32))
            return outs

        pp_cw = attn_proj_chunk(lax.rem(my + HALF, N_DEV))
        pp_ccw = attn_proj_chunk(lax.rem(my - (HALF - 1) + N_DEV, N_DEV))
        pp_own = None
        for h in range(HALF):
            for b in range(B):
                if h == 0:
                    cw_sb[b] = pp_cw[b].astype(_BF16)
                else:
                    cw_sb[b] = (cw_rb[(h - 1) % 2, b].astype(_F32)
                                + pp_cw[b]).astype(_BF16)
            r_cw = pltpu.make_async_remote_copy(
                src_ref=cw_sb, dst_ref=cw_rb.at[h % 2],
                send_sem=rs_cw_s.at[h % 2], recv_sem=rs_cw_r.at[h % 2],
                device_id=(right,), device_id_type=pl.DeviceIdType.MESH)
            r_cw.start()
            if h < HALF - 1:
                for b in range(B):
                    if h == 0:
                        ccw_sb[b] = pp_ccw[b].astype(_BF16)
                    else:
                        ccw_sb[b] = (ccw_rb[(h - 1) % 2, b].astype(_F32)
                                     + pp_ccw[b]).astype(_BF16)
                r_ccw = pltpu.make_async_remote_copy(
                    src_ref=ccw_sb, dst_ref=ccw_rb.at[h % 2],
                    send_sem=rs_ccw_s.at[h % 2], recv_sem=rs_ccw_r.at[h % 2],
                    device_id=(left,), device_id_type=pl.DeviceIdType.MESH)
                r_ccw.start()
            if h < HALF - 1:
                pp_cw = attn_proj_chunk(lax.rem(my + HALF - h - 1 + N_DEV,
                                                N_DEV))
            if h < HALF - 2:
                pp_ccw = attn_proj_chunk(lax.rem(my - (HALF - 1) + h + 1
                                                 + N_DEV, N_DEV))
            if h == HALF - 1:
                pp_own = attn_proj_chunk(my)
            r_cw.wait()
            if h < HALF - 1:
                r_ccw.wait()

        for b in range(B):
            out_ref[b] = (cw_rb[(HALF - 1) % 2, b].astype(_F32)
                          + ccw_rb[(HALF - 2) % 2, b].astype(_F32)
                          + pp_own[b])

    return pl.pallas_call(
        body,
        out_shape=jax.ShapeDtypeStruct((B, SQS, D), _F32),
        in_specs=[pl.BlockSpec(memory_space=pltpu.VMEM)] * 5,
        out_specs=pl.BlockSpec(memory_space=pltpu.VMEM),
        scratch_shapes=[
            pltpu.VMEM((HALF + 1, B, SQS, D), _BF16),
            pltpu.VMEM((HALF, B, SQS, D), _BF16),
            pltpu.VMEM((SQ, DH), _F32),
            pltpu.VMEM((SQ, DH), _F32),
            pltpu.VMEM((B, SQ, HD), _BF16),
            pltpu.VMEM((B, SQ, HD), _BF16),
            pltpu.VMEM((B, SQ, HD), _BF16),
            pltpu.VMEM((D, HD), _BF16),
            pltpu.VMEM((D, HD), _BF16),
            pltpu.VMEM((D, HD), _BF16),
            pltpu.VMEM((HD, D), _BF16),
            pltpu.VMEM((B, SQS, D), _BF16),
            pltpu.VMEM((B, SQS, D), _BF16),
            pltpu.VMEM((2, B, SQS, D), _BF16),
            pltpu.VMEM((2, B, SQS, D), _BF16),
            pltpu.SemaphoreType.DMA((2,)),
            pltpu.SemaphoreType.DMA((2,)),
            pltpu.SemaphoreType.DMA((2,)),
            pltpu.SemaphoreType.DMA((2,)),
            pltpu.SemaphoreType.DMA((2,)),
            pltpu.SemaphoreType.DMA((2,)),
            pltpu.SemaphoreType.DMA((2,)),
            pltpu.SemaphoreType.DMA((2,)),
        ],
        compiler_params=pltpu.CompilerParams(
            collective_id=0, vmem_limit_bytes=100 * 1024 * 1024),
    )(x, Wq, Wk, Wv, Wo)


# device time: 144928 ns/iter; 2.0679x vs baseline; 1.1471x over previous
import jax
import jax.numpy as jnp
from jax import lax
from jax.experimental import pallas as pl
from jax.experimental.pallas import tpu as pltpu

N_DEV = 16
HALF = N_DEV // 2
B = 2
SQS = 128
SQ = N_DEV * SQS
D = 512
H_LOC = 4
DH = 64
HD = H_LOC * DH
_F32 = jnp.float32
_BF16 = jnp.bfloat16


def kernel(x, Wq, Wk, Wv, Wo):
    def body(x_ref, wq_ref, wk_ref, wv_ref, wo_ref, out_ref,
             cw, ccw, cos_ref, sin_ref, qs, ks, vs,
             wqb, wkb, wvb, wob,
             cw_sb, ccw_sb, cw_rb, ccw_rb,
             ag_cw_s, ag_cw_r, ag_ccw_s, ag_ccw_r,
             rs_cw_s, rs_cw_r, rs_ccw_s, rs_ccw_r):
        my = lax.axis_index("i")
        left = lax.rem(my + N_DEV - 1, N_DEV)
        right = lax.rem(my + 1, N_DEV)

        barrier_sem = pltpu.get_barrier_semaphore()
        for nbr in (left, right):
            pl.semaphore_signal(barrier_sem, inc=1, device_id=(nbr,),
                                device_id_type=pl.DeviceIdType.MESH)
        pl.semaphore_wait(barrier_sem, 2)

        posf = lax.broadcasted_iota(jnp.int32, (SQ, DH), 0).astype(_F32)
        di = lax.broadcasted_iota(jnp.int32, (SQ, DH), 1)
        expo = (di - di % 2).astype(_F32) * (1.0 / DH)
        angle = posf * jnp.exp(-jnp.log(10000.0) * expo)
        cos_ref[...] = jnp.cos(angle)
        sin_ref[...] = jnp.sin(angle)

        ii = lax.broadcasted_iota(jnp.int32, (HD, HD), 0)
        jj = lax.broadcasted_iota(jnp.int32, (HD, HD), 1)
        R = (jnp.where((ii == jj - 1) & (jj % 2 == 1), 1.0, 0.0)
             + jnp.where((ii == jj + 1) & (jj % 2 == 0), -1.0, 0.0))

        wqb[...] = wq_ref[...].astype(_BF16)
        wkb[...] = wk_ref[...].astype(_BF16)
        wvb[...] = wv_ref[...].astype(_BF16)
        wob[...] = wo_ref[...].astype(_BF16)

        def qkv_chunks(parts):
            n = len(parts)
            starts = [origin * SQS for _, _, origin in parts]
            cos1 = jnp.concatenate(
                [cos_ref[pl.ds(s, SQS), :] for s in starts], axis=0)
            sin1 = jnp.concatenate(
                [sin_ref[pl.ds(s, SQS), :] for s in starts], axis=0)
            cosc = jnp.concatenate([cos1] * H_LOC, axis=1)
            sinc = jnp.concatenate([sin1] * H_LOC, axis=1)
            for b in range(B):
                xcb = jnp.concatenate(
                    [ref[r, b] for ref, r, _ in parts], axis=0)
                qc = jnp.dot(xcb, wqb[...], preferred_element_type=_F32)
                kc = jnp.dot(xcb, wkb[...], preferred_element_type=_F32)
                vc = jnp.dot(xcb, wvb[...], preferred_element_type=_F32)
                qr = (qc * cosc + jnp.dot(
                    qc, R, preferred_element_type=_F32) * sinc).astype(_BF16)
                kr = (kc * cosc + jnp.dot(
                    kc, R, preferred_element_type=_F32) * sinc).astype(_BF16)
                vr = vc.astype(_BF16)
                for i in range(n):
                    qs[b, pl.ds(starts[i], SQS), :] = qr[i*SQS:(i+1)*SQS]
                    ks[b, pl.ds(starts[i], SQS), :] = kr[i*SQS:(i+1)*SQS]
                    vs[b, pl.ds(starts[i], SQS), :] = vr[i*SQS:(i+1)*SQS]

        cw[0] = x_ref[...].astype(_BF16)
        ccw[0] = cw[0]
        for h in range(HALF):
            r_cw = pltpu.make_async_remote_copy(
                src_ref=cw.at[h], dst_ref=cw.at[h + 1],
                send_sem=ag_cw_s.at[h % 2], recv_sem=ag_cw_r.at[h % 2],
                device_id=(right,), device_id_type=pl.DeviceIdType.MESH)
            r_cw.start()
            if h < HALF - 1:
                r_ccw = pltpu.make_async_remote_copy(
                    src_ref=ccw.at[h], dst_ref=ccw.at[h + 1],
                    send_sem=ag_ccw_s.at[h % 2], recv_sem=ag_ccw_r.at[h % 2],
                    device_id=(left,), device_id_type=pl.DeviceIdType.MESH)
                r_ccw.start()
            if h == 0:
                qkv_chunks([(cw, 0, my)])
            else:
                qkv_chunks([(cw, h, lax.rem(my - h + N_DEV, N_DEV)),
                            (ccw, h, lax.rem(my + h, N_DEV))])
            r_cw.wait()
            if h < HALF - 1:
                r_ccw.wait()
        qkv_chunks([(cw, HALF, lax.rem(my - HALF + N_DEV, N_DEV))])

        def attn_proj(cs):
            n = len(cs)
            outs = [[] for _ in cs]
            for b in range(B):
                ccs = []
                for h in range(H_LOC):
                    qq = jnp.concatenate(
                        [qs[b, pl.ds(c * SQS, SQS), h * DH:(h + 1) * DH]
                         for c in cs], axis=0)
                    kbh = ks[b, :, h * DH:(h + 1) * DH]
                    vbh = vs[b, :, h * DH:(h + 1) * DH]
                    s = lax.dot_general(
                        qq, kbh, (((1,), (1,)), ((), ())),
                        preferred_element_type=_F32) * 0.125
                    m = jnp.max(s, axis=1, keepdims=True)
                    w = jnp.exp(s - m)
                    w = (w / jnp.sum(w, axis=1, keepdims=True)).astype(_BF16)
                    ccs.append(jnp.dot(w, vbh, preferred_element_type=_F32))
                ctxb = jnp.concatenate(ccs, axis=1).astype(_BF16)
                pp = jnp.dot(ctxb, wob[...], preferred_element_type=_F32)
                for i in range(n):
                    outs[i].append(pp[i * SQS:(i + 1) * SQS])
            return outs

        pp_cw, pp_ccw = attn_proj([lax.rem(my + HALF, N_DEV),
                                   lax.rem(my - (HALF - 1) + N_DEV, N_DEV)])
        pp_own = None
        for h in range(HALF):
            for b in range(B):
                if h == 0:
                    cw_sb[b] = pp_cw[b].astype(_BF16)
                else:
                    cw_sb[b] = (cw_rb[(h - 1) % 2, b].astype(_F32)
                                + pp_cw[b]).astype(_BF16)
            r_cw = pltpu.make_async_remote_copy(
                src_ref=cw_sb, dst_ref=cw_rb.at[h % 2],
                send_sem=rs_cw_s.at[h % 2], recv_sem=rs_cw_r.at[h % 2],
                device_id=(right,), device_id_type=pl.DeviceIdType.MESH)
            r_cw.start()
            if h < HALF - 1:
                for b in range(B):
                    if h == 0:
                        ccw_sb[b] = pp_ccw[b].astype(_BF16)
                    else:
                        ccw_sb[b] = (ccw_rb[(h - 1) % 2, b].astype(_F32)
                                     + pp_ccw[b]).astype(_BF16)
                r_ccw = pltpu.make_async_remote_copy(
                    src_ref=ccw_sb, dst_ref=ccw_rb.at[h % 2],
                    send_sem=rs_ccw_s.at[h % 2], recv_sem=rs_ccw_r.at[h % 2],
                    device_id=(left,), device_id_type=pl.DeviceIdType.MESH)
                r_ccw.start()
            if h < HALF - 2:
                pp_cw, pp_ccw = attn_proj(
                    [lax.rem(my + HALF - h - 1 + N_DEV, N_DEV),
                     lax.rem(my - (HALF - 1) + h + 1 + N_DEV, N_DEV)])
            elif h == HALF - 2:
                pp_cw, = attn_proj([lax.rem(my + 1, N_DEV)])
            elif h == HALF - 1:
                pp_own, = attn_proj([my])
            r_cw.wait()
            if h < HALF - 1:
                r_ccw.wait()

        for b in range(B):
            out_ref[b] = (cw_rb[(HALF - 1) % 2, b].astype(_F32)
                          + ccw_rb[(HALF - 2) % 2, b].astype(_F32)
                          + pp_own[b])

    return pl.pallas_call(
        body,
        out_shape=jax.ShapeDtypeStruct((B, SQS, D), _F32),
        in_specs=[pl.BlockSpec(memory_space=pltpu.VMEM)] * 5,
        out_specs=pl.BlockSpec(memory_space=pltpu.VMEM),
        scratch_shapes=[
            pltpu.VMEM((HALF + 1, B, SQS, D), _BF16),
            pltpu.VMEM((HALF, B, SQS, D), _BF16),
            pltpu.VMEM((SQ, DH), _F32),
            pltpu.VMEM((SQ, DH), _F32),
            pltpu.VMEM((B, SQ, HD), _BF16),
            pltpu.VMEM((B, SQ, HD), _BF16),
            pltpu.VMEM((B, SQ, HD), _BF16),
            pltpu.VMEM((D, HD), _BF16),
            pltpu.VMEM((D, HD), _BF16),
            pltpu.VMEM((D, HD), _BF16),
            pltpu.VMEM((HD, D), _BF16),
            pltpu.VMEM((B, SQS, D), _BF16),
            pltpu.VMEM((B, SQS, D), _BF16),
            pltpu.VMEM((2, B, SQS, D), _BF16),
            pltpu.VMEM((2, B, SQS, D), _BF16),
            pltpu.SemaphoreType.DMA((2,)),
            pltpu.SemaphoreType.DMA((2,)),
            pltpu.SemaphoreType.DMA((2,)),
            pltpu.SemaphoreType.DMA((2,)),
            pltpu.SemaphoreType.DMA((2,)),
            pltpu.SemaphoreType.DMA((2,)),
            pltpu.SemaphoreType.DMA((2,)),
            pltpu.SemaphoreType.DMA((2,)),
        ],
        compiler_params=pltpu.CompilerParams(
            collective_id=0, vmem_limit_bytes=100 * 1024 * 1024),
    )(x, Wq, Wk, Wv, Wo)


# device time: 113411 ns/iter; 2.6426x vs baseline; 1.2779x over previous
import jax
import jax.numpy as jnp
from jax import lax
from jax.experimental import pallas as pl
from jax.experimental.pallas import tpu as pltpu

N_DEV = 16
HALF = N_DEV // 2
B = 2
SQS = 128
SQ = N_DEV * SQS
D = 512
H_LOC = 4
DH = 64
HD = H_LOC * DH
_F32 = jnp.float32
_BF16 = jnp.bfloat16


def kernel(x, Wq, Wk, Wv, Wo):
    def body(x_ref, wq_ref, wk_ref, wv_ref, wo_ref, out_ref,
             cw, ccw, cos_ref, sin_ref, qs, ks, vs,
             wqb, wkb, wvb, wob,
             cw_sb, ccw_sb, cw_rb, ccw_rb,
             ag_cw_s, ag_cw_r, ag_ccw_s, ag_ccw_r,
             rs_cw_s, rs_cw_r, rs_ccw_s, rs_ccw_r):
        my = lax.axis_index("i")

        def ring_to_mesh(rp):
            t = rp // 4
            zz = jnp.where(t % 2 == 0, rp % 4, 3 - rp % 4)
            return 4 * zz + (4 - t) % 4

        p_my = my % 4
        t_my = (4 - p_my) % 4
        rp = 4 * t_my + jnp.where(t_my % 2 == 0, my // 4, 3 - my // 4)
        left = ring_to_mesh(lax.rem(rp + N_DEV - 1, N_DEV))
        right = ring_to_mesh(lax.rem(rp + 1, N_DEV))

        barrier_sem = pltpu.get_barrier_semaphore()
        for nbr in (left, right):
            pl.semaphore_signal(barrier_sem, inc=1, device_id=(nbr,),
                                device_id_type=pl.DeviceIdType.MESH)
        pl.semaphore_wait(barrier_sem, 2)

        posf = lax.broadcasted_iota(jnp.int32, (SQ, DH), 0).astype(_F32)
        di = lax.broadcasted_iota(jnp.int32, (SQ, DH), 1)
        expo = (di - di % 2).astype(_F32) * (1.0 / DH)
        angle = posf * jnp.exp(-jnp.log(10000.0) * expo)
        cos_ref[...] = jnp.cos(angle)
        sin_ref[...] = jnp.sin(angle)

        ii = lax.broadcasted_iota(jnp.int32, (HD, HD), 0)
        jj = lax.broadcasted_iota(jnp.int32, (HD, HD), 1)
        R = (jnp.where((ii == jj - 1) & (jj % 2 == 1), 1.0, 0.0)
             + jnp.where((ii == jj + 1) & (jj % 2 == 0), -1.0, 0.0))

        wqb[...] = wq_ref[...].astype(_BF16)
        wkb[...] = wk_ref[...].astype(_BF16)
        wvb[...] = wv_ref[...].astype(_BF16)
        wob[...] = wo_ref[...].astype(_BF16)

        def qkv_chunks(parts):
            n = len(parts)
            starts = [origin * SQS for _, _, origin in parts]
            cos1 = jnp.concatenate(
                [cos_ref[pl.ds(s, SQS), :] for s in starts], axis=0)
            sin1 = jnp.concatenate(
                [sin_ref[pl.ds(s, SQS), :] for s in starts], axis=0)
            cosc = jnp.concatenate([cos1] * H_LOC, axis=1)
            sinc = jnp.concatenate([sin1] * H_LOC, axis=1)
            for b in range(B):
                xcb = jnp.concatenate(
                    [ref[r, b] for ref, r, _ in parts], axis=0)
                qc = jnp.dot(xcb, wqb[...], preferred_element_type=_F32)
                kc = jnp.dot(xcb, wkb[...], preferred_element_type=_F32)
                vc = jnp.dot(xcb, wvb[...], preferred_element_type=_F32)
                qr = (qc * cosc + jnp.dot(
                    qc, R, preferred_element_type=_F32) * sinc).astype(_BF16)
                kr = (kc * cosc + jnp.dot(
                    kc, R, preferred_element_type=_F32) * sinc).astype(_BF16)
                vr = vc.astype(_BF16)
                for i in range(n):
                    qs[b, pl.ds(starts[i], SQS), :] = qr[i*SQS:(i+1)*SQS]
                    ks[b, pl.ds(starts[i], SQS), :] = kr[i*SQS:(i+1)*SQS]
                    vs[b, pl.ds(starts[i], SQS), :] = vr[i*SQS:(i+1)*SQS]

        cw[0] = x_ref[...].astype(_BF16)
        ccw[0] = cw[0]
        for h in range(HALF):
            r_cw = pltpu.make_async_remote_copy(
                src_ref=cw.at[h], dst_ref=cw.at[h + 1],
                send_sem=ag_cw_s.at[h % 2], recv_sem=ag_cw_r.at[h % 2],
                device_id=(right,), device_id_type=pl.DeviceIdType.MESH)
            r_cw.start()
            if h < HALF - 1:
                r_ccw = pltpu.make_async_remote_copy(
                    src_ref=ccw.at[h], dst_ref=ccw.at[h + 1],
                    send_sem=ag_ccw_s.at[h % 2], recv_sem=ag_ccw_r.at[h % 2],
                    device_id=(left,), device_id_type=pl.DeviceIdType.MESH)
                r_ccw.start()
            if h == 0:
                qkv_chunks([(cw, 0, my)])
            else:
                qkv_chunks([
                    (cw, h, ring_to_mesh(lax.rem(rp - h + N_DEV, N_DEV))),
                    (ccw, h, ring_to_mesh(lax.rem(rp + h, N_DEV)))])
            r_cw.wait()
            if h < HALF - 1:
                r_ccw.wait()
        qkv_chunks([(cw, HALF, ring_to_mesh(lax.rem(rp + HALF, N_DEV)))])

        def attn_proj(cs):
            n = len(cs)
            outs = [[] for _ in cs]
            for b in range(B):
                ccs = []
                for h in range(H_LOC):
                    qq = jnp.concatenate(
                        [qs[b, pl.ds(c * SQS, SQS), h * DH:(h + 1) * DH]
                         for c in cs], axis=0)
                    kbh = ks[b, :, h * DH:(h + 1) * DH]
                    vbh = vs[b, :, h * DH:(h + 1) * DH]
                    s = lax.dot_general(
                        qq, kbh, (((1,), (1,)), ((), ())),
                        preferred_element_type=_F32) * 0.125
                    w = jnp.exp(s)
                    w = (w / jnp.sum(w, axis=1, keepdims=True)).astype(_BF16)
                    ccs.append(jnp.dot(w, vbh, preferred_element_type=_F32))
                ctxb = jnp.concatenate(ccs, axis=1).astype(_BF16)
                pp = jnp.dot(ctxb, wob[...], preferred_element_type=_F32)
                for i in range(n):
                    outs[i].append(pp[i * SQS:(i + 1) * SQS])
            return outs

        pp_cw, pp_ccw = attn_proj(
            [ring_to_mesh(lax.rem(rp + HALF, N_DEV)),
             ring_to_mesh(lax.rem(rp - (HALF - 1) + N_DEV, N_DEV))])
        pp_own = None
        for h in range(HALF):
            for b in range(B):
                if h == 0:
                    cw_sb[b] = pp_cw[b].astype(_BF16)
                else:
                    cw_sb[b] = (cw_rb[(h - 1) % 2, b].astype(_F32)
                                + pp_cw[b]).astype(_BF16)
            r_cw = pltpu.make_async_remote_copy(
                src_ref=cw_sb, dst_ref=cw_rb.at[h % 2],
                send_sem=rs_cw_s.at[h % 2], recv_sem=rs_cw_r.at[h % 2],
                device_id=(right,), device_id_type=pl.DeviceIdType.MESH)
            r_cw.start()
            if h < HALF - 1:
                for b in range(B):
                    if h == 0:
                        ccw_sb[b] = pp_ccw[b].astype(_BF16)
                    else:
                        ccw_sb[b] = (ccw_rb[(h - 1) % 2, b].astype(_F32)
                                     + pp_ccw[b]).astype(_BF16)
                r_ccw = pltpu.make_async_remote_copy(
                    src_ref=ccw_sb, dst_ref=ccw_rb.at[h % 2],
                    send_sem=rs_ccw_s.at[h % 2], recv_sem=rs_ccw_r.at[h % 2],
                    device_id=(left,), device_id_type=pl.DeviceIdType.MESH)
                r_ccw.start()
            if h < HALF - 2:
                pp_cw, pp_ccw = attn_proj(
                    [ring_to_mesh(lax.rem(rp + HALF - h - 1 + N_DEV, N_DEV)),
                     ring_to_mesh(lax.rem(rp - (HALF - 1) + h + 1 + N_DEV,
                                          N_DEV))])
            elif h == HALF - 2:
                pp_cw, = attn_proj([ring_to_mesh(lax.rem(rp + 1, N_DEV))])
            elif h == HALF - 1:
                pp_own, = attn_proj([my])
            r_cw.wait()
            if h < HALF - 1:
                r_ccw.wait()

        for b in range(B):
            out_ref[b] = (cw_rb[(HALF - 1) % 2, b].astype(_F32)
                          + ccw_rb[(HALF - 2) % 2, b].astype(_F32)
                          + pp_own[b])

    return pl.pallas_call(
        body,
        out_shape=jax.ShapeDtypeStruct((B, SQS, D), _F32),
        in_specs=[pl.BlockSpec(memory_space=pltpu.VMEM)] * 5,
        out_specs=pl.BlockSpec(memory_space=pltpu.VMEM),
        scratch_shapes=[
            pltpu.VMEM((HALF + 1, B, SQS, D), _BF16),
            pltpu.VMEM((HALF, B, SQS, D), _BF16),
            pltpu.VMEM((SQ, DH), _F32),
            pltpu.VMEM((SQ, DH), _F32),
            pltpu.VMEM((B, SQ, HD), _BF16),
            pltpu.VMEM((B, SQ, HD), _BF16),
            pltpu.VMEM((B, SQ, HD), _BF16),
            pltpu.VMEM((D, HD), _BF16),
            pltpu.VMEM((D, HD), _BF16),
            pltpu.VMEM((D, HD), _BF16),
            pltpu.VMEM((HD, D), _BF16),
            pltpu.VMEM((B, SQS, D), _BF16),
            pltpu.VMEM((B, SQS, D), _BF16),
            pltpu.VMEM((2, B, SQS, D), _BF16),
            pltpu.VMEM((2, B, SQS, D), _BF16),
            pltpu.SemaphoreType.DMA((2,)),
            pltpu.SemaphoreType.DMA((2,)),
            pltpu.SemaphoreType.DMA((2,)),
            pltpu.SemaphoreType.DMA((2,)),
            pltpu.SemaphoreType.DMA((2,)),
            pltpu.SemaphoreType.DMA((2,)),
            pltpu.SemaphoreType.DMA((2,)),
            pltpu.SemaphoreType.DMA((2,)),
        ],
        compiler_params=pltpu.CompilerParams(
            collective_id=0, vmem_limit_bytes=100 * 1024 * 1024),
    )(x, Wq, Wk, Wv, Wo)


# device time: 99226 ns/iter; 3.0204x vs baseline; 1.1430x over previous
import jax
import jax.numpy as jnp
from jax import lax
from jax.experimental import pallas as pl
from jax.experimental.pallas import tpu as pltpu

N_DEV = 16
HALF = N_DEV // 2
B = 2
SQS = 128
SQ = N_DEV * SQS
D = 512
H_LOC = 4
DH = 64
HD = H_LOC * DH
_F32 = jnp.float32
_BF16 = jnp.bfloat16


def kernel(x, Wq, Wk, Wv, Wo):
    def body(x_ref, wq_ref, wk_ref, wv_ref, wo_ref, out_ref,
             cw, ccw, cos_ref, sin_ref, qs, ks, vs,
             wqb, wkb, wvb, wob,
             cw_sb, ccw_sb, cw_rb, ccw_rb,
             ag_cw_s, ag_cw_r, ag_ccw_s, ag_ccw_r,
             rs_cw_s, rs_cw_r, rs_ccw_s, rs_ccw_r):
        my = lax.axis_index("i")

        def ring_to_mesh(rp):
            t = rp // 4
            zz = jnp.where(t % 2 == 0, rp % 4, 3 - rp % 4)
            return 4 * zz + (4 - t) % 4

        p_my = my % 4
        t_my = (4 - p_my) % 4
        rp = 4 * t_my + jnp.where(t_my % 2 == 0, my // 4, 3 - my // 4)
        left = ring_to_mesh(lax.rem(rp + N_DEV - 1, N_DEV))
        right = ring_to_mesh(lax.rem(rp + 1, N_DEV))

        barrier_sem = pltpu.get_barrier_semaphore()
        for nbr in (left, right):
            pl.semaphore_signal(barrier_sem, inc=1, device_id=(nbr,),
                                device_id_type=pl.DeviceIdType.MESH)
        pl.semaphore_wait(barrier_sem, 2)

        posf = lax.broadcasted_iota(jnp.int32, (SQ, DH), 0).astype(_F32)
        di = lax.broadcasted_iota(jnp.int32, (SQ, DH), 1)
        expo = (di - di % 2).astype(_F32) * (1.0 / DH)
        angle = posf * jnp.exp(-jnp.log(10000.0) * expo)
        cos_ref[...] = jnp.cos(angle)
        sin_ref[...] = jnp.sin(angle)

        ii = lax.broadcasted_iota(jnp.int32, (HD, HD), 0)
        jj = lax.broadcasted_iota(jnp.int32, (HD, HD), 1)
        R = (jnp.where((ii == jj - 1) & (jj % 2 == 1), 1.0, 0.0)
             + jnp.where((ii == jj + 1) & (jj % 2 == 0), -1.0, 0.0))

        wqb[...] = wq_ref[...].astype(_BF16)
        wkb[...] = wk_ref[...].astype(_BF16)
        wvb[...] = wv_ref[...].astype(_BF16)
        wob[...] = wo_ref[...].astype(_BF16)

        def qkv_chunks(parts):
            n = len(parts)
            starts = [origin * SQS for _, _, origin in parts]
            cos1 = jnp.concatenate(
                [cos_ref[pl.ds(s, SQS), :] for s in starts], axis=0)
            sin1 = jnp.concatenate(
                [sin_ref[pl.ds(s, SQS), :] for s in starts], axis=0)
            cosc = jnp.concatenate([cos1] * H_LOC, axis=1)
            sinc = jnp.concatenate([sin1] * H_LOC, axis=1)
            for b in range(B):
                xcb = jnp.concatenate(
                    [ref[r, b] for ref, r, _ in parts], axis=0)
                qc = jnp.dot(xcb, wqb[...], preferred_element_type=_F32)
                kc = jnp.dot(xcb, wkb[...], preferred_element_type=_F32)
                vc = jnp.dot(xcb, wvb[...], preferred_element_type=_F32)
                qr = (qc * cosc + jnp.dot(
                    qc, R, preferred_element_type=_F32) * sinc).astype(_BF16)
                kr = (kc * cosc + jnp.dot(
                    kc, R, preferred_element_type=_F32) * sinc).astype(_BF16)
                vr = vc.astype(_BF16)
                for i in range(n):
                    qs[b, pl.ds(starts[i], SQS), :] = qr[i*SQS:(i+1)*SQS]
                    ks[b, pl.ds(starts[i], SQS), :] = kr[i*SQS:(i+1)*SQS]
                    vs[b, pl.ds(starts[i], SQS), :] = vr[i*SQS:(i+1)*SQS]

        cw[0] = x_ref[...].astype(_BF16)
        ccw[0] = cw[0]
        for h in range(HALF):
            r_cw = pltpu.make_async_remote_copy(
                src_ref=cw.at[h], dst_ref=cw.at[h + 1],
                send_sem=ag_cw_s.at[h % 2], recv_sem=ag_cw_r.at[h % 2],
                device_id=(right,), device_id_type=pl.DeviceIdType.MESH)
            r_cw.start()
            if h < HALF - 1:
                r_ccw = pltpu.make_async_remote_copy(
                    src_ref=ccw.at[h], dst_ref=ccw.at[h + 1],
                    send_sem=ag_ccw_s.at[h % 2], recv_sem=ag_ccw_r.at[h % 2],
                    device_id=(left,), device_id_type=pl.DeviceIdType.MESH)
                r_ccw.start()
            if h == 0:
                qkv_chunks([(cw, 0, my)])
            else:
                qkv_chunks([
                    (cw, h, ring_to_mesh(lax.rem(rp - h + N_DEV, N_DEV))),
                    (ccw, h, ring_to_mesh(lax.rem(rp + h, N_DEV)))])
            r_cw.wait()
            if h < HALF - 1:
                r_ccw.wait()
        qkv_chunks([(cw, HALF, ring_to_mesh(lax.rem(rp + HALF, N_DEV)))])

        def attn_proj(cs):
            n = len(cs)
            outs = [[] for _ in cs]
            for b in range(B):
                ccs = []
                for h in range(H_LOC):
                    qq = jnp.concatenate(
                        [qs[b, pl.ds(c * SQS, SQS), h * DH:(h + 1) * DH]
                         for c in cs], axis=0) * 0.125
                    kbh = ks[b, :, h * DH:(h + 1) * DH]
                    vbh = vs[b, :, h * DH:(h + 1) * DH]
                    s = lax.dot_general(
                        qq, kbh, (((1,), (1,)), ((), ())),
                        preferred_element_type=_F32)
                    w = jnp.exp(s)
                    denom = jnp.sum(w, axis=1, keepdims=True)
                    cc = jnp.dot(w.astype(_BF16), vbh,
                                 preferred_element_type=_F32)
                    ccs.append(cc / denom)
                ctxb = jnp.concatenate(ccs, axis=1).astype(_BF16)
                pp = jnp.dot(ctxb, wob[...], preferred_element_type=_F32)
                for i in range(n):
                    outs[i].append(pp[i * SQS:(i + 1) * SQS])
            return outs

        pp_cw, pp_ccw = attn_proj(
            [ring_to_mesh(lax.rem(rp + HALF, N_DEV)),
             ring_to_mesh(lax.rem(rp - (HALF - 1) + N_DEV, N_DEV))])
        pp_own = None
        for h in range(HALF):
            for b in range(B):
                if h == 0:
                    cw_sb[b] = pp_cw[b].astype(_BF16)
                else:
                    cw_sb[b] = (cw_rb[(h - 1) % 2, b].astype(_F32)
                                + pp_cw[b]).astype(_BF16)
            r_cw = pltpu.make_async_remote_copy(
                src_ref=cw_sb, dst_ref=cw_rb.at[h % 2],
                send_sem=rs_cw_s.at[h % 2], recv_sem=rs_cw_r.at[h % 2],
                device_id=(right,), device_id_type=pl.DeviceIdType.MESH)
            r_cw.start()
            if h < HALF - 1:
                for b in range(B):
                    if h == 0:
                        ccw_sb[b] = pp_ccw[b].astype(_BF16)
                    else:
                        ccw_sb[b] = (ccw_rb[(h - 1) % 2, b].astype(_F32)
                                     + pp_ccw[b]).astype(_BF16)
                r_ccw = pltpu.make_async_remote_copy(
                    src_ref=ccw_sb, dst_ref=ccw_rb.at[h % 2],
                    send_sem=rs_ccw_s.at[h % 2], recv_sem=rs_ccw_r.at[h % 2],
                    device_id=(left,), device_id_type=pl.DeviceIdType.MESH)
                r_ccw.start()
            if h < HALF - 2:
                pp_cw, pp_ccw = attn_proj(
                    [ring_to_mesh(lax.rem(rp + HALF - h - 1 + N_DEV, N_DEV)),
                     ring_to_mesh(lax.rem(rp - (HALF - 1) + h + 1 + N_DEV,
                                          N_DEV))])
            elif h == HALF - 2:
                pp_cw, = attn_proj([ring_to_mesh(lax.rem(rp + 1, N_DEV))])
            elif h == HALF - 1:
                pp_own, = attn_proj([my])
            r_cw.wait()
            if h < HALF - 1:
                r_ccw.wait()

        for b in range(B):
            out_ref[b] = (cw_rb[(HALF - 1) % 2, b].astype(_F32)
                          + ccw_rb[(HALF - 2) % 2, b].astype(_F32)
                          + pp_own[b])

    return pl.pallas_call(
        body,
        out_shape=jax.ShapeDtypeStruct((B, SQS, D), _F32),
        in_specs=[pl.BlockSpec(memory_space=pltpu.VMEM)] * 5,
        out_specs=pl.BlockSpec(memory_space=pltpu.VMEM),
        scratch_shapes=[
            pltpu.VMEM((HALF + 1, B, SQS, D), _BF16),
            pltpu.VMEM((HALF, B, SQS, D), _BF16),
            pltpu.VMEM((SQ, DH), _F32),
            pltpu.VMEM((SQ, DH), _F32),
            pltpu.VMEM((B, SQ, HD), _BF16),
            pltpu.VMEM((B, SQ, HD), _BF16),
            pltpu.VMEM((B, SQ, HD), _BF16),
            pltpu.VMEM((D, HD), _BF16),
            pltpu.VMEM((D, HD), _BF16),
            pltpu.VMEM((D, HD), _BF16),
            pltpu.VMEM((HD, D), _BF16),
            pltpu.VMEM((B, SQS, D), _BF16),
            pltpu.VMEM((B, SQS, D), _BF16),
            pltpu.VMEM((2, B, SQS, D), _BF16),
            pltpu.VMEM((2, B, SQS, D), _BF16),
            pltpu.SemaphoreType.DMA((2,)),
            pltpu.SemaphoreType.DMA((2,)),
            pltpu.SemaphoreType.DMA((2,)),
            pltpu.SemaphoreType.DMA((2,)),
            pltpu.SemaphoreType.DMA((2,)),
            pltpu.SemaphoreType.DMA((2,)),
            pltpu.SemaphoreType.DMA((2,)),
            pltpu.SemaphoreType.DMA((2,)),
        ],
        compiler_params=pltpu.CompilerParams(
            collective_id=0, vmem_limit_bytes=100 * 1024 * 1024),
    )(x, Wq, Wk, Wv, Wo)
